# Initial kernel scaffold; baseline (speedup 1.0000x reference)
import jax
import jax.numpy as jnp
from jax.experimental import pallas as pl


def kernel(x, x_positions, slopes, biases):
    raise NotImplementedError("write your pallas kernel here")


# speedup vs baseline: 11.0747x; 11.0747x over previous
import functools

import jax
import jax.numpy as jnp
from jax import lax
from jax.experimental import pallas as pl
from jax.experimental.pallas import tpu as pltpu
from jax.experimental.pallas import tpu_sc as plsc

_K = 100
_TPAD = 128


def _build_tables(x_positions, slopes, biases):
    sx = jnp.sort(x_positions[0])
    skips = jnp.roll(sx, -1) - sx
    skip_deltas = skips * slopes[0, 1:]
    cums = jnp.cumsum(skip_deltas)[:-1]
    b0 = biases[0]
    beta = jnp.concatenate([b0[None], b0[None], cums + b0])
    bp = jnp.concatenate([sx[:1], sx])
    sl = slopes[0]
    t_pad = jnp.concatenate(
        [sx, jnp.full((_TPAD - _K,), jnp.inf, jnp.float32)])
    zpad = jnp.zeros((_TPAD - (_K + 1),), jnp.float32)
    return (t_pad,
            jnp.concatenate([bp, zpad]),
            jnp.concatenate([beta, zpad]),
            jnp.concatenate([sl, zpad]))


@functools.lru_cache(maxsize=None)
def _make_pwl_map(n):
    info = plsc.get_sparse_core_info()
    nc, ns, nl = info.num_cores, info.num_subcores, info.num_lanes
    nw = nc * ns
    assert n % (nw * nl) == 0
    n_per_w = n // nw
    mesh = plsc.VectorSubcoreMesh(core_axis_name="c", subcore_axis_name="s")

    @functools.partial(
        pl.kernel,
        mesh=mesh,
        compiler_params=pltpu.CompilerParams(needs_layout_passes=False),
        out_type=jax.ShapeDtypeStruct((n,), jnp.float32),
        scratch_types=[
            pltpu.VMEM((n_per_w,), jnp.float32),
            pltpu.VMEM((n_per_w,), jnp.float32),
            pltpu.VMEM((_TPAD,), jnp.float32),
            pltpu.VMEM((_TPAD,), jnp.float32),
            pltpu.VMEM((_TPAD,), jnp.float32),
            pltpu.VMEM((_TPAD,), jnp.float32),
        ],
    )
    def pwl_map(x_hbm, t_hbm, bp_hbm, beta_hbm, sl_hbm, out_hbm,
                xv, yv, tv, bpv, betav, slv):
        wid = lax.axis_index("s") * nc + lax.axis_index("c")
        base = wid * n_per_w
        pltpu.sync_copy(t_hbm, tv)
        pltpu.sync_copy(bp_hbm, bpv)
        pltpu.sync_copy(beta_hbm, betav)
        pltpu.sync_copy(sl_hbm, slv)
        pltpu.sync_copy(x_hbm.at[pl.ds(base, n_per_w)], xv)

        def body(i, carry):
            v = xv[pl.ds(i * nl, nl)]
            b = jnp.zeros((nl,), jnp.int32)
            for step in (64, 32, 16, 8, 4, 2, 1):
                probe = b + step
                t_probe = plsc.load_gather(tv, [probe - 1])
                b = jnp.where(t_probe <= v, probe, b)
            bp = plsc.load_gather(bpv, [b])
            beta = plsc.load_gather(betav, [b])
            sl = plsc.load_gather(slv, [b])
            y = beta + (v - bp) * sl
            yv[pl.ds(i * nl, nl)] = jnp.clip(y, 0.0, 1.0)
            return carry

        lax.fori_loop(0, n_per_w // nl, body, 0, unroll=4)
        pltpu.sync_copy(yv, out_hbm.at[pl.ds(base, n_per_w)])

    return pwl_map


def kernel(x, x_positions, slopes, biases):
    t_pad, bp_t, beta_t, sl_t = _build_tables(x_positions, slopes, biases)
    x_flat = x.reshape(-1)
    y_flat = _make_pwl_map(x_flat.shape[0])(x_flat, t_pad, bp_t, beta_t, sl_t)
    return (y_flat.reshape(x.shape),)


# speedup vs baseline: 14.3435x; 1.2952x over previous
import functools

import jax
import jax.numpy as jnp
from jax import lax
from jax.experimental import pallas as pl
from jax.experimental.pallas import tpu as pltpu
from jax.experimental.pallas import tpu_sc as plsc

_K = 100
_TPAD = 128


def _build_tables(x_positions, slopes, biases):
    sx = jnp.sort(x_positions[0])
    skips = jnp.roll(sx, -1) - sx
    skip_deltas = skips * slopes[0, 1:]
    cums = jnp.cumsum(skip_deltas)[:-1]
    b0 = biases[0]
    beta = jnp.concatenate([b0[None], b0[None], cums + b0])
    bp = jnp.concatenate([sx[:1], sx])
    sl = slopes[0]
    t_pad = jnp.concatenate(
        [sx, jnp.full((_TPAD - _K,), jnp.inf, jnp.float32)])
    root = t_pad[7::8]
    zpad = jnp.zeros((_TPAD - (_K + 1),), jnp.float32)
    a_t = jnp.concatenate([beta - bp * sl, zpad])
    s_t = jnp.concatenate([sl, zpad])
    return t_pad, root, a_t, s_t


@functools.lru_cache(maxsize=None)
def _make_pwl_map(n):
    info = plsc.get_sparse_core_info()
    nc, ns, nl = info.num_cores, info.num_subcores, info.num_lanes
    nw = nc * ns
    assert n % (nw * nl) == 0
    n_per_w = n // nw
    mesh = plsc.VectorSubcoreMesh(core_axis_name="c", subcore_axis_name="s")

    @functools.partial(
        pl.kernel,
        mesh=mesh,
        compiler_params=pltpu.CompilerParams(needs_layout_passes=False),
        out_type=jax.ShapeDtypeStruct((n,), jnp.float32),
        scratch_types=[
            pltpu.VMEM((n_per_w,), jnp.float32),
            pltpu.VMEM((n_per_w,), jnp.float32),
            pltpu.VMEM((_TPAD,), jnp.float32),
            pltpu.VMEM((16,), jnp.float32),
            pltpu.VMEM((_TPAD,), jnp.float32),
            pltpu.VMEM((_TPAD,), jnp.float32),
        ],
    )
    def pwl_map(x_hbm, t_hbm, root_hbm, a_hbm, s_hbm, out_hbm,
                xv, yv, tv, rootv, av, sv):
        wid = lax.axis_index("s") * nc + lax.axis_index("c")
        base = wid * n_per_w
        pltpu.sync_copy(t_hbm, tv)
        pltpu.sync_copy(root_hbm, rootv)
        pltpu.sync_copy(a_hbm, av)
        pltpu.sync_copy(s_hbm, sv)
        pltpu.sync_copy(x_hbm.at[pl.ds(base, n_per_w)], xv)
        root = rootv[...]

        def body(i, carry):
            v = xv[pl.ds(i * nl, nl)]
            b = jnp.zeros((nl,), jnp.int32)
            for step in (64, 32, 16, 8):
                probe = b + step
                t_probe = jnp.take_along_axis(
                    root, jax.lax.shift_right_logical(probe, 3) - 1,
                    axis=0, mode="promise_in_bounds")
                b = jnp.where(t_probe <= v, probe, b)
            for step in (4, 2, 1):
                probe = b + step
                t_probe = plsc.load_gather(tv, [probe - 1])
                b = jnp.where(t_probe <= v, probe, b)
            a = plsc.load_gather(av, [b])
            s = plsc.load_gather(sv, [b])
            yv[pl.ds(i * nl, nl)] = jnp.clip(a + v * s, 0.0, 1.0)
            return carry

        lax.fori_loop(0, n_per_w // nl, body, 0, unroll=8)
        pltpu.sync_copy(yv, out_hbm.at[pl.ds(base, n_per_w)])

    return pwl_map


def kernel(x, x_positions, slopes, biases):
    t_pad, root, a_t, s_t = _build_tables(x_positions, slopes, biases)
    x_flat = x.reshape(-1)
    y_flat = _make_pwl_map(x_flat.shape[0])(x_flat, t_pad, root, a_t, s_t)
    return (y_flat.reshape(x.shape),)


# speedup vs baseline: 25.5084x; 1.7784x over previous
import functools

import jax
import jax.numpy as jnp
from jax import lax
from jax.experimental import pallas as pl
from jax.experimental.pallas import tpu as pltpu
from jax.experimental.pallas import tpu_sc as plsc

_K = 100
_TPAD = 128


def _build_tables(x_positions, slopes, biases):
    sx = jnp.sort(x_positions[0])
    skips = jnp.roll(sx, -1) - sx
    skip_deltas = skips * slopes[0, 1:]
    cums = jnp.cumsum(skip_deltas)[:-1]
    b0 = biases[0]
    beta = jnp.concatenate([b0[None], b0[None], cums + b0])
    bp = jnp.concatenate([sx[:1], sx])
    sl = slopes[0]
    t_pad = jnp.concatenate(
        [sx, jnp.full((_TPAD - _K,), jnp.inf, jnp.float32)])
    root = t_pad[7::8]
    zpad = jnp.zeros((_TPAD - (_K + 1),), jnp.float32)
    a_t = jnp.concatenate([beta - bp * sl, zpad])
    s_t = jnp.concatenate([sl, zpad])
    return t_pad, root, a_t, s_t


@functools.lru_cache(maxsize=None)
def _make_pwl_map(n):
    info = plsc.get_sparse_core_info()
    nc, ns, nl = info.num_cores, info.num_subcores, info.num_lanes
    nw = nc * ns
    assert n % (nw * nl) == 0
    n_per_w = n // nw
    mesh = plsc.VectorSubcoreMesh(core_axis_name="c", subcore_axis_name="s")

    @functools.partial(
        pl.kernel,
        mesh=mesh,
        compiler_params=pltpu.CompilerParams(needs_layout_passes=False),
        out_type=jax.ShapeDtypeStruct((n,), jnp.float32),
        scratch_types=[
            pltpu.VMEM((n_per_w,), jnp.float32),
            pltpu.VMEM((n_per_w,), jnp.float32),
            pltpu.VMEM((_TPAD,), jnp.float32),
            pltpu.VMEM((16,), jnp.float32),
            pltpu.VMEM((_TPAD,), jnp.float32),
            pltpu.VMEM((_TPAD,), jnp.float32),
        ],
    )
    def pwl_map(x_hbm, t_hbm, root_hbm, a_hbm, s_hbm, out_hbm,
                xv, yv, tv, rootv, av, sv):
        wid = lax.axis_index("s") * nc + lax.axis_index("c")
        base = wid * n_per_w
        pltpu.sync_copy(t_hbm, tv)
        pltpu.sync_copy(root_hbm, rootv)
        pltpu.sync_copy(a_hbm, av)
        pltpu.sync_copy(s_hbm, sv)
        pltpu.sync_copy(x_hbm.at[pl.ds(base, n_per_w)], xv)
        root = rootv[...]

        @plsc.parallel_loop(0, n_per_w, nl, unroll=8)
        def body(i):
            v = xv[pl.ds(i, nl)]
            b = jnp.zeros((nl,), jnp.int32)
            for step in (64, 32, 16, 8):
                probe = b + step
                t_probe = jnp.take_along_axis(
                    root, jax.lax.shift_right_logical(probe, 3) - 1,
                    axis=0, mode="promise_in_bounds")
                b = jnp.where(t_probe <= v, probe, b)
            for step in (4, 2, 1):
                probe = b + step
                t_probe = plsc.load_gather(tv, [probe - 1])
                b = jnp.where(t_probe <= v, probe, b)
            a = plsc.load_gather(av, [b])
            s = plsc.load_gather(sv, [b])
            yv[pl.ds(i, nl)] = jnp.clip(a + v * s, 0.0, 1.0)

        pltpu.sync_copy(yv, out_hbm.at[pl.ds(base, n_per_w)])

    return pwl_map


def kernel(x, x_positions, slopes, biases):
    t_pad, root, a_t, s_t = _build_tables(x_positions, slopes, biases)
    x_flat = x.reshape(-1)
    y_flat = _make_pwl_map(x_flat.shape[0])(x_flat, t_pad, root, a_t, s_t)
    return (y_flat.reshape(x.shape),)


# speedup vs baseline: 28.1521x; 1.1036x over previous
import functools

import jax
import jax.numpy as jnp
from jax import lax
from jax.experimental import pallas as pl
from jax.experimental.pallas import tpu as pltpu
from jax.experimental.pallas import tpu_sc as plsc

_K = 100
_KPAD = 112
_TPAD = 128


@functools.lru_cache(maxsize=None)
def _make_pwl_map(n):
    info = plsc.get_sparse_core_info()
    nc, ns, nl = info.num_cores, info.num_subcores, info.num_lanes
    nw = nc * ns
    assert n % (nw * nl) == 0 and nl == 16
    n_per_w = n // nw
    nchunk = _KPAD // nl
    mesh = plsc.VectorSubcoreMesh(core_axis_name="c", subcore_axis_name="s")

    @functools.partial(
        pl.kernel,
        mesh=mesh,
        compiler_params=pltpu.CompilerParams(needs_layout_passes=False),
        out_type=jax.ShapeDtypeStruct((n,), jnp.float32),
        scratch_types=[
            pltpu.VMEM((n_per_w,), jnp.float32),
            pltpu.VMEM((n_per_w,), jnp.float32),
            pltpu.VMEM((_KPAD,), jnp.float32),
            pltpu.VMEM((_TPAD,), jnp.float32),
            pltpu.VMEM((_TPAD,), jnp.float32),
            pltpu.VMEM((_KPAD,), jnp.float32),
            pltpu.VMEM((nl,), jnp.float32),
        ],
    )
    def pwl_map(x_hbm, tu_hbm, sl_hbm, b_hbm, out_hbm,
                xv, yv, tuv, tv, av, slv, bv):
        wid = lax.axis_index("s") * nc + lax.axis_index("c")
        base = wid * n_per_w
        pltpu.sync_copy(tu_hbm, tuv)
        pltpu.sync_copy(sl_hbm, slv)
        pltpu.sync_copy(b_hbm, bv)
        pltpu.sync_copy(x_hbm.at[pl.ds(base, n_per_w)], xv)

        iota = jnp.arange(nl, dtype=jnp.int32)
        inf16 = jnp.full((nl,), jnp.inf, jnp.float32)

        w = [tuv[pl.ds(c * nl, nl)] for c in range(nchunk)]

        def rank_body(j, ranks):
            bc = plsc.load_gather(tuv, [jnp.full((nl,), j, jnp.int32)])
            out = []
            for c in range(nchunk):
                lt = bc < w[c]
                eq_before = (bc == w[c]) & (j < (iota + c * nl))
                out.append(ranks[c] + jnp.where(lt | eq_before, 1, 0))
            return tuple(out)

        ranks = lax.fori_loop(
            0, _K, rank_body,
            tuple(jnp.zeros((nl,), jnp.int32) for _ in range(nchunk)),
            unroll=4)

        tv[pl.ds(_TPAD - 2 * nl, nl)] = inf16
        tv[pl.ds(_TPAD - nl, nl)] = inf16
        for c in range(nchunk):
            plsc.store_scatter(tv, [ranks[c]], w[c],
                               mask=(iota + c * nl) < _K)

        b0 = bv[...]
        carry = jnp.zeros((nl,), jnp.float32)
        for c in range(nchunk):
            sx = tv[pl.ds(c * nl, nl)]
            sxm1 = plsc.load_gather(
                tv, [jnp.maximum(iota + (c * nl - 1), 0)])
            s = slv[pl.ds(c * nl, nl)]
            dd = (sx - sxm1) * s
            cum = plsc.cumsum(dd) + carry
            shifted = jnp.take_along_axis(
                cum, jnp.maximum(iota - 1, 0), axis=0,
                mode="promise_in_bounds")
            beta = b0 + jnp.where(iota == 0, carry, shifted)
            av[pl.ds(c * nl, nl)] = beta - sxm1 * s
            carry = jnp.take_along_axis(
                cum, jnp.full((nl,), nl - 1, jnp.int32), axis=0,
                mode="promise_in_bounds")

        root = plsc.load_gather(tv, [iota * 8 + 7])

        @plsc.parallel_loop(0, n_per_w, nl, unroll=8)
        def body(i):
            v = xv[pl.ds(i, nl)]
            b = jnp.zeros((nl,), jnp.int32)
            for step in (64, 32, 16, 8):
                probe = b + step
                t_probe = jnp.take_along_axis(
                    root, jax.lax.shift_right_logical(probe, 3) - 1,
                    axis=0, mode="promise_in_bounds")
                b = jnp.where(t_probe <= v, probe, b)
            for step in (4, 2, 1):
                probe = b + step
                t_probe = plsc.load_gather(tv, [probe - 1])
                b = jnp.where(t_probe <= v, probe, b)
            a = plsc.load_gather(av, [b])
            s = plsc.load_gather(slv, [b])
            yv[pl.ds(i, nl)] = jnp.clip(a + v * s, 0.0, 1.0)

        pltpu.sync_copy(yv, out_hbm.at[pl.ds(base, n_per_w)])

    return pwl_map


def kernel(x, x_positions, slopes, biases):
    tu_pad = jnp.concatenate(
        [x_positions[0], jnp.full((_KPAD - _K,), jnp.inf, jnp.float32)])
    sl_pad = jnp.concatenate(
        [slopes[0], jnp.zeros((_KPAD - (_K + 1),), jnp.float32)])
    b16 = jnp.broadcast_to(biases, (16,))
    x_flat = x.reshape(-1)
    y_flat = _make_pwl_map(x_flat.shape[0])(x_flat, tu_pad, sl_pad, b16)
    return (y_flat.reshape(x.shape),)
